# baseline (speedup 1.0000x reference)
import functools

import jax
import jax.numpy as jnp
from jax import lax
from jax.experimental import pallas as pl
from jax.experimental.pallas import tpu as pltpu
from jax.experimental.pallas import tpu_sc as plsc

_INFO = plsc.get_sparse_core_info()
_NC = _INFO.num_cores
_NS = _INFO.num_subcores
_NW = _NC * _NS
_L = _INFO.num_lanes

_B = 16384
_D = 64
_PATHS = 64
_N = 1000000
_HROWS = _N // 2
_HBLK = _HROWS // 2
_BPW = _B // _NW
_CH = 128
_NCHUNK = _BPW // _CH
_CGROUPS = _CH // _L

_mesh = plsc.VectorSubcoreMesh(core_axis_name="c", subcore_axis_name="s")


@functools.partial(
    pl.kernel,
    out_type=jax.ShapeDtypeStruct((_B,), jnp.float32),
    mesh=_mesh,
    compiler_params=pltpu.CompilerParams(needs_layout_passes=False),
    scratch_types=[
        pltpu.VMEM((_BPW,), jnp.int32),
        pltpu.VMEM((_BPW,), jnp.int32),
        pltpu.VMEM((_BPW,), jnp.int32),
        pltpu.VMEM((2, _NCHUNK, _CH), jnp.int32),
        pltpu.VMEM((2, _NCHUNK, _CH), jnp.int32),
        pltpu.VMEM((2, _CH, 2 * _D), jnp.float32),
        pltpu.VMEM((2, _CH, 2 * _D), jnp.float32),
        pltpu.VMEM((_PATHS * _D,), jnp.float32),
        pltpu.VMEM((_BPW,), jnp.float32),
        pltpu.SemaphoreType.DMA,
    ],
)
def _hin2vec_sc(start_hbm, end_hbm, path_hbm, ntab0_hbm, ntab1_hbm,
                ptabf_hbm, out_hbm,
                sidx_v, eidx_v, path_v, sblk_v, eblk_v, srows_v, erows_v,
                ptab_v, out_v, sem):
    wid = lax.axis_index("s") * _NC + lax.axis_index("c")
    base = wid * _BPW

    pltpu.sync_copy(start_hbm.at[pl.ds(base, _BPW)], sidx_v)
    pltpu.sync_copy(end_hbm.at[pl.ds(base, _BPW)], eidx_v)
    pltpu.sync_copy(path_hbm.at[pl.ds(base, _BPW)], path_v)
    pltpu.sync_copy(ptabf_hbm, ptab_v)

    maxb = jnp.full((_L,), _HBLK - 1, jnp.int32)
    zero = jnp.zeros((_L,), jnp.int32)
    for j in range(_NCHUNK):
        for g in range(_CH // _L):
            svec = sidx_v[pl.ds(j * _CH + g * _L, _L)] >> 1
            evec = eidx_v[pl.ds(j * _CH + g * _L, _L)] >> 1
            sblk_v[0, j, pl.ds(g * _L, _L)] = jnp.minimum(svec, maxb)
            sblk_v[1, j, pl.ds(g * _L, _L)] = jnp.maximum(svec - _HBLK, zero)
            eblk_v[0, j, pl.ds(g * _L, _L)] = jnp.minimum(evec, maxb)
            eblk_v[1, j, pl.ds(g * _L, _L)] = jnp.maximum(evec - _HBLK, zero)

    halves = (ntab0_hbm, ntab1_hbm)

    lane = lax.broadcasted_iota(jnp.int32, (_L,), 0)

    for j in range(_NCHUNK):
        descs = []
        for h in range(2):
            descs.append(pltpu.async_copy(
                halves[h].at[sblk_v.at[h, j]], srows_v.at[h], sem))
            descs.append(pltpu.async_copy(
                halves[h].at[eblk_v.at[h, j]], erows_v.at[h], sem))
        for d_ in descs:
            d_.wait()

        @pl.loop(0, _CGROUPS)
        def group_body(g):
            row_idx = g * _L + lane
            svec = sidx_v[pl.ds(j * _CH + g * _L, _L)]
            evec = eidx_v[pl.ds(j * _CH + g * _L, _L)]
            shalf = (svec >= _HROWS).astype(jnp.int32)
            ehalf = (evec >= _HROWS).astype(jnp.int32)
            scol = (svec & 1) * _D
            ecol = (evec & 1) * _D
            path_g = path_v[pl.ds(j * _CH + g * _L, _L)]
            pathbase = path_g * _D

            @pl.loop(0, _D, init_carry=jnp.zeros((_L,), jnp.float32),
                     unroll=8)
            def dim_body(d, acc):
                s_g = plsc.load_gather(srows_v, [shalf, row_idx, scol + d])
                e_g = plsc.load_gather(erows_v, [ehalf, row_idx, ecol + d])
                p_g = plsc.load_gather(ptab_v, [pathbase + d])
                return acc + jnp.where(p_g >= 0.0, s_g * e_g, 0.0)

            acc = dim_body
            out_v[pl.ds(j * _CH + g * _L, _L)] = 1.0 / (1.0 + jnp.exp(-acc))

    pltpu.sync_copy(out_v, out_hbm.at[pl.ds(base, _BPW)])


def kernel(start_node, end_node, path, node_table, path_table):
    ntab0 = node_table[:_HROWS].reshape(_HBLK, 2 * _D)
    ntab1 = node_table[_HROWS:].reshape(_HBLK, 2 * _D)
    return _hin2vec_sc(start_node.astype(jnp.int32), end_node.astype(jnp.int32),
                       path.astype(jnp.int32), ntab0, ntab1,
                       path_table.reshape(-1))


# speedup vs baseline: 3.9295x; 3.9295x over previous
import functools

import jax
import jax.numpy as jnp
from jax import lax
from jax.experimental import pallas as pl
from jax.experimental.pallas import tpu as pltpu
from jax.experimental.pallas import tpu_sc as plsc

_INFO = plsc.get_sparse_core_info()
_NC = _INFO.num_cores
_NS = _INFO.num_subcores
_NW = _NC * _NS
_L = _INFO.num_lanes

_B = 16384
_D = 64
_PATHS = 64
_BPW = _B // _NW
_CHUNK = 128
_NCHUNK = _BPW // _CHUNK
_CGROUPS = _CHUNK // _L

_mesh = plsc.VectorSubcoreMesh(core_axis_name="c", subcore_axis_name="s")


@functools.partial(
    pl.kernel,
    out_type=jax.ShapeDtypeStruct((_B,), jnp.float32),
    mesh=_mesh,
    compiler_params=pltpu.CompilerParams(needs_layout_passes=False),
    scratch_types=[
        pltpu.VMEM((_BPW,), jnp.int32),
        pltpu.VMEM((_BPW,), jnp.int32),
        pltpu.VMEM((_BPW,), jnp.int32),
        pltpu.VMEM((2, _CHUNK, _D), jnp.float32),
        pltpu.VMEM((2, _CHUNK, _D), jnp.float32),
        pltpu.VMEM((_PATHS * _D,), jnp.float32),
        pltpu.VMEM((_BPW,), jnp.float32),
        pltpu.SemaphoreType.DMA,
        pltpu.SemaphoreType.DMA,
    ],
)
def _hin2vec_sc(start_hbm, end_hbm, path_hbm, ntab_hbm, ptabf_hbm, out_hbm,
                sidx_v, eidx_v, path_v, srows_v, erows_v, ptab_v, out_v,
                sem0, sem1):
    wid = lax.axis_index("s") * _NC + lax.axis_index("c")
    base = wid * _BPW

    pltpu.sync_copy(start_hbm.at[pl.ds(base, _BPW)], sidx_v)
    pltpu.sync_copy(end_hbm.at[pl.ds(base, _BPW)], eidx_v)
    pltpu.sync_copy(path_hbm.at[pl.ds(base, _BPW)], path_v)
    pltpu.sync_copy(ptabf_hbm, ptab_v)

    sems = (sem0, sem1)
    lane = lax.broadcasted_iota(jnp.int32, (_L,), 0)

    def fire(c):
        buf = c % 2
        cbase = c * _CHUNK

        @pl.loop(0, _CHUNK // _L)
        def fetch(b):
            sivec = sidx_v[pl.ds(cbase + b * _L, _L)]
            eivec = eidx_v[pl.ds(cbase + b * _L, _L)]
            for k in range(_L):
                pltpu.async_copy(ntab_hbm.at[sivec[k]],
                                 srows_v.at[buf, b * _L + k], sems[buf])
                pltpu.async_copy(ntab_hbm.at[eivec[k]],
                                 erows_v.at[buf, b * _L + k], sems[buf])

    def drain(c):
        buf = c % 2
        pltpu.make_async_copy(
            ntab_hbm.at[pl.ds(0, _CHUNK)], srows_v.at[buf], sems[buf]).wait()
        pltpu.make_async_copy(
            ntab_hbm.at[pl.ds(0, _CHUNK)], erows_v.at[buf], sems[buf]).wait()

    fire(0)
    for c in range(_NCHUNK):
        if c + 1 < _NCHUNK:
            fire(c + 1)
        drain(c)
        buf = c % 2
        cbase = c * _CHUNK
        srows = srows_v.at[buf]
        erows = erows_v.at[buf]

        @pl.loop(0, _CGROUPS)
        def group_body(g):
            row_idx = g * _L + lane
            path_g = path_v[pl.ds(cbase + g * _L, _L)]
            pathbase = path_g * _D

            @pl.loop(0, _D, init_carry=jnp.zeros((_L,), jnp.float32),
                     unroll=8)
            def dim_body(d, acc):
                dvec = jnp.broadcast_to(d, (_L,)).astype(jnp.int32)
                s_g = plsc.load_gather(srows, [row_idx, dvec])
                e_g = plsc.load_gather(erows, [row_idx, dvec])
                p_g = plsc.load_gather(ptab_v, [pathbase + d])
                return acc + jnp.where(p_g >= 0.0, s_g * e_g, 0.0)

            acc = dim_body
            out_v[pl.ds(cbase + g * _L, _L)] = 1.0 / (1.0 + jnp.exp(-acc))

    pltpu.sync_copy(out_v, out_hbm.at[pl.ds(base, _BPW)])


def kernel(start_node, end_node, path, node_table, path_table):
    return _hin2vec_sc(start_node.astype(jnp.int32), end_node.astype(jnp.int32),
                       path.astype(jnp.int32), node_table,
                       path_table.reshape(-1))
